# Initial kernel scaffold; baseline (speedup 1.0000x reference)
import jax
import jax.numpy as jnp
from jax.experimental import pallas as pl


def kernel(x, types, indexs, attn_vector):
    raise NotImplementedError("write your pallas kernel here")


# speedup vs baseline: 16.3924x; 16.3924x over previous
import math

import jax
import jax.numpy as jnp
from jax.experimental import pallas as pl

_NUM_SEG = 256


def _body(x_ref, tcol_ref, irow_ref, icol_ref, av_ref, o_ref):
    n, hd = x_ref.shape[1], x_ref.shape[2]
    t = av_ref.shape[0]
    h = o_ref.shape[2]
    d = hd // h
    s = _NUM_SEG

    xv = x_ref[0]
    tcol = tcol_ref[0]
    irow = irow_ref[0]
    icol = icol_ref[0]
    av = av_ref[...]

    oh_t = (tcol == jax.lax.broadcasted_iota(jnp.int32, (n, t), 1)).astype(jnp.float32)
    per_tok = jax.lax.dot_general(oh_t, av, (((1,), (0,)), ((), ())),
                                  preferred_element_type=jnp.float32)
    prod = xv * per_tok

    r = jax.lax.broadcasted_iota(jnp.int32, (hd, h), 0) // d
    c = jax.lax.broadcasted_iota(jnp.int32, (hd, h), 1)
    sel = (r == c).astype(jnp.float32)
    attns = jax.lax.dot_general(prod, sel, (((1,), (0,)), ((), ())),
                                preferred_element_type=jnp.float32)
    attns = attns * (1.0 / math.sqrt(d))

    gmax = jnp.max(attns, axis=0, keepdims=True)
    e = jnp.exp(attns - gmax)

    oh_sT = (irow == jax.lax.broadcasted_iota(jnp.int32, (s, n), 0)).astype(jnp.float32)
    oh_s = (icol == jax.lax.broadcasted_iota(jnp.int32, (n, s), 1)).astype(jnp.float32)
    ssum = jax.lax.dot_general(oh_sT, e, (((1,), (0,)), ((), ())),
                               preferred_element_type=jnp.float32)
    ssum_g = jax.lax.dot_general(oh_s, ssum, (((1,), (0,)), ((), ())),
                                 preferred_element_type=jnp.float32)

    o_ref[0] = e / (ssum_g + 1e-16)


def kernel(x, types, indexs, attn_vector):
    b, n, h, d = x.shape
    t = attn_vector.shape[0]
    hd = h * d

    x2 = x.reshape(b, n, hd)
    tcol = types.reshape(b, n, 1).astype(jnp.int32)
    irow = indexs.reshape(b, 1, n).astype(jnp.int32)
    icol = indexs.reshape(b, n, 1).astype(jnp.int32)
    av2 = attn_vector.reshape(t, hd)

    out = pl.pallas_call(
        _body,
        grid=(b,),
        in_specs=[
            pl.BlockSpec((1, n, hd), lambda i: (i, 0, 0)),
            pl.BlockSpec((1, n, 1), lambda i: (i, 0, 0)),
            pl.BlockSpec((1, 1, n), lambda i: (i, 0, 0)),
            pl.BlockSpec((1, n, 1), lambda i: (i, 0, 0)),
            pl.BlockSpec((t, hd), lambda i: (0, 0)),
        ],
        out_specs=pl.BlockSpec((1, n, h), lambda i: (i, 0, 0)),
        out_shape=jax.ShapeDtypeStruct((b, n, h), jnp.float32),
    )(x2, tcol, irow, icol, av2)
    return out


# speedup vs baseline: 17.0893x; 1.0425x over previous
import math

import jax
import jax.numpy as jnp
from jax.experimental import pallas as pl

_NUM_SEG = 256


def _body(x_ref, tcol_ref, irow_ref, icol_ref, w_ref, o_ref):
    n, hd = x_ref.shape[1], x_ref.shape[2]
    h = o_ref.shape[2]
    th = w_ref.shape[1]
    s = _NUM_SEG

    xv = x_ref[0]
    tcol = tcol_ref[0]
    irow = irow_ref[0]
    icol = icol_ref[0]
    w = w_ref[...]

    all_sc = jax.lax.dot_general(xv, w, (((1,), (0,)), ((), ())),
                                 preferred_element_type=jnp.float32)
    lane_t = jax.lax.broadcasted_iota(jnp.int32, (n, th), 1) // h
    masked = jnp.where(lane_t == tcol, all_sc, 0.0)

    kmod = jax.lax.broadcasted_iota(jnp.int32, (th, h), 0) % h
    hidx = jax.lax.broadcasted_iota(jnp.int32, (th, h), 1)
    sel2 = (kmod == hidx).astype(jnp.float32)
    attns = jax.lax.dot_general(masked, sel2, (((1,), (0,)), ((), ())),
                                preferred_element_type=jnp.float32)
    attns = attns * (1.0 / math.sqrt(hd // h))

    gmax = jnp.max(attns, axis=0, keepdims=True)
    e = jnp.exp(attns - gmax)

    oh_sT = (irow == jax.lax.broadcasted_iota(jnp.int32, (s, n), 0)).astype(jnp.float32)
    oh_s = (icol == jax.lax.broadcasted_iota(jnp.int32, (n, s), 1)).astype(jnp.float32)
    ssum = jax.lax.dot_general(oh_sT, e, (((1,), (0,)), ((), ())),
                               preferred_element_type=jnp.float32)
    ssum_g = jax.lax.dot_general(oh_s, ssum, (((1,), (0,)), ((), ())),
                                 preferred_element_type=jnp.float32)

    o_ref[0] = e / (ssum_g + 1e-16)


def kernel(x, types, indexs, attn_vector):
    b, n, h, d = x.shape
    t = attn_vector.shape[0]
    hd = h * d

    x2 = x.reshape(b, n, hd)
    tcol = types.reshape(b, n, 1).astype(jnp.int32)
    irow = indexs.reshape(b, 1, n).astype(jnp.int32)
    icol = indexs.reshape(b, n, 1).astype(jnp.int32)

    av3 = jnp.transpose(attn_vector[:, 0], (1, 2, 0))
    w = (av3[:, :, :, None] * jnp.eye(h, dtype=x.dtype)[:, None, None, :])
    w = w.reshape(hd, t * h)

    out = pl.pallas_call(
        _body,
        grid=(b,),
        in_specs=[
            pl.BlockSpec((1, n, hd), lambda i: (i, 0, 0)),
            pl.BlockSpec((1, n, 1), lambda i: (i, 0, 0)),
            pl.BlockSpec((1, 1, n), lambda i: (i, 0, 0)),
            pl.BlockSpec((1, n, 1), lambda i: (i, 0, 0)),
            pl.BlockSpec((hd, t * h), lambda i: (0, 0)),
        ],
        out_specs=pl.BlockSpec((1, n, h), lambda i: (i, 0, 0)),
        out_shape=jax.ShapeDtypeStruct((b, n, h), jnp.float32),
    )(x2, tcol, irow, icol, w)
    return out
